# baseline (speedup 1.0000x reference)
import jax
import jax.numpy as jnp
from jax import lax
from jax.experimental import pallas as pl

C1 = 10
C2 = 1000
BATCH = 16384
BBc = 1024


def _loss_body(p1_ref, x_ref, tab_ref, out_ref):
    x = x_ref[...]
    m = jnp.max(x, axis=0, keepdims=True)
    lse = m + jnp.log(jnp.sum(jnp.exp(x - m), axis=0, keepdims=True))

    p1 = p1_ref[...]
    row = lax.broadcasted_iota(jnp.int32, (C1, BBc), 0)
    pm = jnp.max(p1, axis=0, keepdims=True)
    fi = jnp.min(jnp.where(p1 == pm, row, C1), axis=0, keepdims=True)
    onehot = (row == fi).astype(jnp.float32)
    labels = jnp.dot(tab_ref[...], onehot, preferred_element_type=jnp.float32)
    t = jnp.sum(labels * x, axis=0, keepdims=True)

    partial = (jnp.sum(lse - t) * (1.0 / BATCH)).reshape(1, 1)

    @pl.when(pl.program_id(0) == 0)
    def _():
        out_ref[...] = jnp.zeros((1, 1), jnp.float32)

    out_ref[...] += partial


@jax.jit
def kernel(pred1_logits, pred2_logits, label_table):
    p1t = pred1_logits.T
    xt = pred2_logits.T
    tabt = label_table.T
    out = pl.pallas_call(
        _loss_body,
        grid=(BATCH // BBc,),
        in_specs=[
            pl.BlockSpec((C1, BBc), lambda i: (0, i)),
            pl.BlockSpec((C2, BBc), lambda i: (0, i)),
            pl.BlockSpec((C2, C1), lambda i: (0, 0)),
        ],
        out_specs=pl.BlockSpec((1, 1), lambda i: (0, 0)),
        out_shape=jax.ShapeDtypeStruct((1, 1), jnp.float32),
    )(p1t, xt, tabt)
    return out[0, 0]


# speedup vs baseline: 1.1324x; 1.1324x over previous
import jax
import jax.numpy as jnp
from jax import lax
from jax.experimental import pallas as pl

C1 = 10
C2 = 1000
BATCH = 16384
BBc = 2048


def _loss_body(p1_ref, x_ref, tab_ref, out_ref):
    x = x_ref[...]
    m = jnp.max(x, axis=0, keepdims=True)
    lse = m + jnp.log(jnp.sum(jnp.exp(x - m), axis=0, keepdims=True))

    p1 = p1_ref[...]
    row = lax.broadcasted_iota(jnp.int32, (C1, BBc), 0)
    pm = jnp.max(p1, axis=0, keepdims=True)
    fi = jnp.min(jnp.where(p1 == pm, row, C1), axis=0, keepdims=True)
    onehot = (row == fi).astype(jnp.float32)
    labels = jnp.dot(tab_ref[...], onehot, preferred_element_type=jnp.float32)
    t = jnp.sum(labels * x, axis=0, keepdims=True)

    partial = (jnp.sum(lse - t) * (1.0 / BATCH)).reshape(1, 1)

    @pl.when(pl.program_id(0) == 0)
    def _():
        out_ref[...] = jnp.zeros((1, 1), jnp.float32)

    out_ref[...] += partial


@jax.jit
def kernel(pred1_logits, pred2_logits, label_table):
    p1t = pred1_logits.T
    xt = pred2_logits.T
    tabt = label_table.T
    out = pl.pallas_call(
        _loss_body,
        grid=(BATCH // BBc,),
        in_specs=[
            pl.BlockSpec((C1, BBc), lambda i: (0, i)),
            pl.BlockSpec((C2, BBc), lambda i: (0, i)),
            pl.BlockSpec((C2, C1), lambda i: (0, 0)),
        ],
        out_specs=pl.BlockSpec((1, 1), lambda i: (0, 0)),
        out_shape=jax.ShapeDtypeStruct((1, 1), jnp.float32),
    )(p1t, xt, tabt)
    return out[0, 0]
